# baseline (speedup 1.0000x reference)
import jax
import jax.numpy as jnp
from jax import lax
from jax.experimental import pallas as pl
from jax.experimental.pallas import tpu as pltpu
from jax.experimental.pallas import tpu_sc as plsc

N = 10000
NI = 320000
D = 128
G = 64
L = 16

NC = 2
NS = 16
NW = NC * NS
CHUNK = 128
N_CHUNKS = 80
PER_TILE = CHUNK * N_CHUNKS
NI_PAD = PER_TILE * NW
NP = 10240
PAD_ROW = N
ROWS_PER_TILE = NP // NS

_MESH = plsc.VectorSubcoreMesh(core_axis_name="c", subcore_axis_name="s")

_f32 = jnp.float32


def _phase_body(table, pairs, zeros_nd, zeros_np,
                out, cnt_out,
                pv0, pv1, pv2, pv3, rv0, rv1, hist,
                acc_sh,
                si0, si1, si2, si3, sg0, sg1, ss0, ss1):
    pair_v = (pv0, pv1, pv2, pv3)
    src_v = tuple(p.at[0] for p in pair_v)
    dst_v = tuple(p.at[1] for p in pair_v)
    rows_v = (rv0, rv1)
    sem_i = (si0, si1, si2, si3)
    sem_g = (sg0, sg1)
    sem_s = (ss0, ss1)

    cid = lax.axis_index("c")
    sid = lax.axis_index("s")
    wid = cid * NS + sid

    r0 = sid * ROWS_PER_TILE
    ones16 = jnp.ones((L,), _f32)

    def issue_idx(j, c):
        pltpu.async_copy(pairs.at[wid * N_CHUNKS + c], pair_v[j], sem_i[j])

    def wait_idx(j):
        pltpu.make_async_copy(pairs.at[0], pair_v[j], sem_i[j]).wait()

    issue_idx(0, 0)
    issue_idx(1, 1)
    pltpu.sync_copy(zeros_nd.at[pl.ds(r0, ROWS_PER_TILE)],
                    acc_sh.at[pl.ds(r0, ROWS_PER_TILE)])
    pltpu.sync_copy(zeros_np, hist)
    plsc.subcore_barrier()

    def issue_scatter(j, b):
        pltpu.async_copy(rows_v[b], acc_sh.at[dst_v[j]], sem_s[b], add=True)

    def wait_scatter(j, b):
        pltpu.make_async_copy(rows_v[b], acc_sh.at[dst_v[j]], sem_s[b]).wait()

    def wait_gather(j, b):
        pltpu.make_async_copy(table.at[src_v[j]], rows_v[b], sem_g[b]).wait()

    def hist_update(j):
        for k in range(CHUNK // L):
            idx16 = pair_v[j][1, pl.ds(k * L, L)]
            plsc.addupdate_scatter(hist, [idx16], ones16)

    def body(s, carry):
        for j in range(4):
            c = 4 * s + j
            b = j % 2
            wait_idx(j)

            @pl.when(c >= 2)
            def _(j=j, b=b):
                wait_scatter((j + 2) % 4, b)

            @pl.when(c + 2 < N_CHUNKS)
            def _(j=j, c=c):
                issue_idx((j + 2) % 4, c + 2)

            pltpu.async_copy(table.at[src_v[j]], rows_v[b], sem_g[b])
            hist_update(j)

            @pl.when(c >= 1)
            def _(j=j, b=b):
                wait_gather((j + 3) % 4, 1 - b)
                issue_scatter((j + 3) % 4, 1 - b)

        return carry

    lax.fori_loop(0, N_CHUNKS // 4, body, 0)

    j_last = (N_CHUNKS - 1) % 4
    b_last = (N_CHUNKS - 1) % 2
    wait_gather(j_last, b_last)
    issue_scatter(j_last, b_last)
    wait_scatter((N_CHUNKS - 2) % 4, (N_CHUNKS - 2) % 2)
    wait_scatter(j_last, b_last)

    plsc.subcore_barrier()

    hist_cp = pltpu.async_copy(hist, cnt_out.at[wid], sem_s[0])
    descs = {}
    for k in range(ROWS_PER_TILE // CHUNK):
        b = k % 2
        if k >= 2:
            descs[k - 2].wait()
        pltpu.sync_copy(acc_sh.at[pl.ds(r0 + k * CHUNK, CHUNK)], rows_v[b])
        descs[k] = pltpu.async_copy(
            rows_v[b], out.at[pl.ds(cid * NP + r0 + k * CHUNK, CHUNK)],
            sem_g[b])
    descs[ROWS_PER_TILE // CHUNK - 2].wait()
    descs[ROWS_PER_TILE // CHUNK - 1].wait()
    hist_cp.wait()


_phase = pl.kernel(
    _phase_body,
    out_type=(
        jax.ShapeDtypeStruct((NC * NP, D), _f32),
        jax.ShapeDtypeStruct((NW, NP), _f32),
    ),
    mesh=_MESH,
    scratch_types=(
        [pltpu.VMEM((2, CHUNK), jnp.int32)] * 4
        + [pltpu.VMEM((CHUNK, D), _f32)] * 2
        + [pltpu.VMEM((NP,), _f32)]
        + [pltpu.VMEM_SHARED((NP, D), _f32)]
        + [pltpu.SemaphoreType.DMA] * 8
    ),
    compiler_params=pltpu.CompilerParams(use_tc_tiling_on_sc=False,
                                         needs_layout_passes=False),
)


_RB = 1000
_NB = N // _RB


def _tc_matmul(x, W):
    def body(x_ref, w_ref, o_ref):
        o_ref[...] = jnp.dot(x_ref[...], w_ref[...],
                             preferred_element_type=_f32)

    return pl.pallas_call(
        body,
        grid=(_NB,),
        in_specs=[pl.BlockSpec((_RB, D), lambda i: (i, 0)),
                  pl.BlockSpec((D, D), lambda i: (0, 0))],
        out_specs=pl.BlockSpec((_RB, D), lambda i: (i, 0)),
        out_shape=jax.ShapeDtypeStruct((N, D), _f32),
    )(x, W)


def _combine_block(p_ref, c_ref):
    s = p_ref[0] + p_ref[1]
    cnt = jnp.sum(c_ref[:, 0, 0, :], axis=0)[:, None]
    inv = jnp.where(cnt > 0.0, 1.0 / cnt, 0.0)
    return s * inv


def _tc_combine(partials, cnts):
    p3 = partials.reshape(NC, NP, D)
    cnts = cnts[:, :N].reshape(NW, _NB, 1, _RB)

    def body(p_ref, c_ref, o_ref):
        o_ref[...] = _combine_block(p_ref, c_ref)

    return pl.pallas_call(
        body,
        grid=(_NB,),
        in_specs=[pl.BlockSpec((NC, _RB, D), lambda i: (0, i, 0)),
                  pl.BlockSpec((NW, 1, 1, _RB), lambda i: (0, i, 0, 0))],
        out_specs=pl.BlockSpec((_RB, D), lambda i: (i, 0)),
        out_shape=jax.ShapeDtypeStruct((N, D), _f32),
    )(p3, cnts)


def _tc_combine_relu_mm(partials, cnts, bias, W):
    p3 = partials.reshape(NC, NP, D)
    cnts = cnts[:, :N].reshape(NW, _NB, 1, _RB)

    def body(p_ref, c_ref, b_ref, w_ref, o_ref):
        h = jnp.maximum(
            _combine_block(p_ref, c_ref) + b_ref[...], 0.0)
        o_ref[...] = jnp.dot(h, w_ref[...], preferred_element_type=_f32)

    return pl.pallas_call(
        body,
        grid=(_NB,),
        in_specs=[pl.BlockSpec((NC, _RB, D), lambda i: (0, i, 0)),
                  pl.BlockSpec((NW, 1, 1, _RB), lambda i: (0, i, 0, 0)),
                  pl.BlockSpec((1, D), lambda i: (0, 0)),
                  pl.BlockSpec((D, D), lambda i: (0, 0))],
        out_specs=pl.BlockSpec((_RB, D), lambda i: (i, 0)),
        out_shape=jax.ShapeDtypeStruct((N, D), _f32),
    )(p3, cnts, bias.reshape(1, D), W)


def _tc_combine_relu_pool(partials, cnts, bias, batch3d):
    p3 = partials.reshape(NC, NP, D)
    cnts = cnts[:, :N].reshape(NW, _NB, 1, _RB)

    def body(p_ref, c_ref, b_ref, bt_ref, o_ref, sums, pcnts):
        i = pl.program_id(0)

        @pl.when(i == 0)
        def _():
            sums[...] = jnp.zeros_like(sums)
            pcnts[...] = jnp.zeros_like(pcnts)

        h = jnp.maximum(_combine_block(p_ref, c_ref) + b_ref[...], 0.0)
        b = bt_ref[0, 0, :]
        mask = (b[:, None] == lax.broadcasted_iota(jnp.int32, (_RB, G), 1)
                ).astype(_f32)
        sums[...] += lax.dot_general(mask, h, (((0,), (0,)), ((), ())),
                                     preferred_element_type=_f32)
        pcnts[...] += jnp.broadcast_to(jnp.sum(mask, axis=0)[:, None], (G, D))

        @pl.when(i == _NB - 1)
        def _():
            o_ref[...] = sums[...] / jnp.maximum(pcnts[...], 1.0)

    return pl.pallas_call(
        body,
        grid=(_NB,),
        in_specs=[pl.BlockSpec((NC, _RB, D), lambda i: (0, i, 0)),
                  pl.BlockSpec((NW, 1, 1, _RB), lambda i: (0, i, 0, 0)),
                  pl.BlockSpec((1, D), lambda i: (0, 0)),
                  pl.BlockSpec((1, 1, _RB), lambda i: (i, 0, 0))],
        out_specs=pl.BlockSpec((G, D), lambda i: (0, 0)),
        out_shape=jax.ShapeDtypeStruct((G, D), _f32),
        scratch_shapes=[pltpu.VMEM((G, D), _f32), pltpu.VMEM((G, D), _f32)],
    )(p3, cnts, bias.reshape(1, D), batch3d)


def kernel(x, hyperedge_index, batch, W1, b1, W2, b2):
    node_idx = hyperedge_index[0].astype(jnp.int32)
    edge_idx = hyperedge_index[1].astype(jnp.int32)
    batch3d = batch.astype(jnp.int32).reshape(_NB, 1, _RB)

    n_pad = NI_PAD - NI
    src_pad = jnp.arange(n_pad, dtype=jnp.int32) % N
    dst_pad = PAD_ROW + (jnp.arange(n_pad, dtype=jnp.int32) % (NP - N))
    node_src = jnp.concatenate([node_idx, src_pad])
    node_dst = jnp.concatenate([node_idx, dst_pad])
    edge_src = jnp.concatenate([edge_idx, src_pad])
    edge_dst = jnp.concatenate([edge_idx, dst_pad])
    pairsA = jnp.stack([node_src.reshape(-1, CHUNK),
                        edge_dst.reshape(-1, CHUNK)], axis=1)
    pairsB = jnp.stack([edge_src.reshape(-1, CHUNK),
                        node_dst.reshape(-1, CHUNK)], axis=1)

    zeros_nd = jnp.zeros((NP, D), _f32)
    zeros_np = jnp.zeros((NP,), _f32)

    xw = _tc_matmul(x, W1)
    heP, cntB = _phase(xw, pairsA, zeros_nd, zeros_np)
    he = _tc_combine(heP, cntB)
    outP, cntD = _phase(he, pairsB, zeros_nd, zeros_np)
    xw = _tc_combine_relu_mm(outP, cntD, b1, W2)

    heP, cntB = _phase(xw, pairsA, zeros_nd, zeros_np)
    he = _tc_combine(heP, cntB)
    outP, cntD = _phase(he, pairsB, zeros_nd, zeros_np)
    return _tc_combine_relu_pool(outP, cntD, b2, batch3d)
